# baseline (speedup 1.0000x reference)
import jax
import jax.numpy as jnp
from jax import lax
from jax.experimental import pallas as pl
from jax.experimental.pallas import tpu as pltpu
from jax.experimental.pallas import tpu_sc as plsc

B, S, D = 4, 4096, 1024
L = 2048
NC, NS = 2, 16
NW = NC * NS
RPW = (B * L) // NW
WPB = L // RPW
GS = 32
G = RPW // GS
NBUF = 3


def _body(x_hbm, idx_hbm, out_hbm, idx_v, *rest):
    bufs = rest[:NBUF]
    gsems = rest[NBUF:2 * NBUF]
    wsems = rest[2 * NBUF:3 * NBUF]
    wid = lax.axis_index("s") * NC + lax.axis_index("c")
    b = wid // WPB
    off = b * S
    base = wid * RPW

    pltpu.sync_copy(idx_hbm.at[b, pl.ds((wid % WPB) * RPW, RPW)], idx_v)

    gd = [None] * NBUF
    wd = [None] * NBUF

    def issue_gather(g):
        nb = g % NBUF
        if wd[nb] is not None:
            wd[nb].wait()
        sub = []
        for h in range(GS // 16):
            v = idx_v[pl.ds(g * GS + h * 16, 16)]
            vm = jnp.maximum(v, 0) + off
            sub.append(pltpu.async_copy(
                x_hbm.at[vm], bufs[nb].at[pl.ds(h * 16, 16)], gsems[nb]))
        gd[nb] = sub

    for g in range(NBUF - 1):
        issue_gather(g)
    for g in range(G):
        cb = g % NBUF
        if g + NBUF - 1 < G:
            issue_gather(g + NBUF - 1)
        for d in gd[cb]:
            d.wait()
        wd[cb] = pltpu.async_copy(
            bufs[cb], out_hbm.at[pl.ds(base + g * GS, GS)], wsems[cb])
    for g in range(G - NBUF, G):
        wd[g % NBUF].wait()


def kernel(x, indices):
    xf = x.reshape(B * S, D)
    idx = indices.astype(jnp.int32)
    mesh = plsc.VectorSubcoreMesh(core_axis_name="c", subcore_axis_name="s")
    out = pl.kernel(
        _body,
        mesh=mesh,
        out_type=jax.ShapeDtypeStruct((B * L, D), jnp.float32),
        scratch_types=(
            [pltpu.VMEM((RPW,), jnp.int32)]
            + [pltpu.VMEM((GS, D), jnp.float32)] * NBUF
            + [pltpu.SemaphoreType.DMA] * (2 * NBUF)
        ),
    )(xf, idx)
    return out.reshape(B, L, D)


# speedup vs baseline: 1.0053x; 1.0053x over previous
import jax
import jax.numpy as jnp
from jax import lax
from jax.experimental import pallas as pl
from jax.experimental.pallas import tpu as pltpu
from jax.experimental.pallas import tpu_sc as plsc

B, S, D = 4, 4096, 1024
L = 2048
NC, NS = 2, 16
NW = NC * NS
RPW = (B * L) // NW
WPB = L // RPW
GS = 16
G = RPW // GS
NBUF = 7


def _body(x_hbm, idx_hbm, out_hbm, idx_v, *rest):
    bufs = rest[:NBUF]
    gsems = rest[NBUF:2 * NBUF]
    wsems = rest[2 * NBUF:3 * NBUF]
    wid = lax.axis_index("s") * NC + lax.axis_index("c")
    b = wid // WPB
    off = b * S
    base = wid * RPW

    pltpu.sync_copy(idx_hbm.at[b, pl.ds((wid % WPB) * RPW, RPW)], idx_v)

    gd = [None] * NBUF
    wd = [None] * NBUF

    def issue_gather(g):
        nb = g % NBUF
        if wd[nb] is not None:
            wd[nb].wait()
        v = idx_v[pl.ds(g * GS, 16)]
        vm = jnp.maximum(v, 0) + off
        gd[nb] = pltpu.async_copy(x_hbm.at[vm], bufs[nb], gsems[nb])

    for g in range(NBUF - 1):
        issue_gather(g)
    for g in range(G):
        cb = g % NBUF
        if g + NBUF - 1 < G:
            issue_gather(g + NBUF - 1)
        gd[cb].wait()
        wd[cb] = pltpu.async_copy(
            bufs[cb], out_hbm.at[pl.ds(base + g * GS, GS)], wsems[cb])
    for g in range(G - NBUF, G):
        wd[g % NBUF].wait()


def kernel(x, indices):
    xf = x.reshape(B * S, D)
    idx = indices.astype(jnp.int32)
    mesh = plsc.VectorSubcoreMesh(core_axis_name="c", subcore_axis_name="s")
    out = pl.kernel(
        _body,
        mesh=mesh,
        out_type=jax.ShapeDtypeStruct((B * L, D), jnp.float32),
        scratch_types=(
            [pltpu.VMEM((RPW,), jnp.int32)]
            + [pltpu.VMEM((GS, D), jnp.float32)] * NBUF
            + [pltpu.SemaphoreType.DMA] * (2 * NBUF)
        ),
    )(xf, idx)
    return out.reshape(B, L, D)
